# baseline (speedup 1.0000x reference)
import functools

import jax
import jax.numpy as jnp
from jax import lax
from jax.experimental import pallas as pl
from jax.experimental.pallas import tpu as pltpu
from jax.experimental.pallas import tpu_sc as plsc

E = 8
D = 2048
F = 4096
T = 8192
P = 2 * T
BT = 512
NT = P // BT
NI = NT + E - 1
NW = 32
GT = 1024


def _gate_body(x_ref, wg_ref, bg_ref, e_ref, w_ref, xi_ref):
    x = x_ref[...].astype(jnp.bfloat16)
    wg = wg_ref[...].astype(jnp.bfloat16)
    logits = lax.dot_general(x, wg, (((1,), (0,)), ((), ())),
                             preferred_element_type=jnp.float32)
    logits = logits + bg_ref[...]
    bits = lax.bitcast_convert_type(x.astype(jnp.float32), jnp.int32)
    lo = lax.shift_right_logical(bits[:, :D // 2], 16)
    hi = jnp.bitwise_and(bits[:, D // 2:], jnp.int32(-65536))
    xi_ref[...] = jnp.bitwise_or(lo, hi)
    idx = lax.broadcasted_iota(jnp.int32, logits.shape, 1)
    m0 = jnp.max(logits, axis=1, keepdims=True)
    e0 = jnp.min(jnp.where(logits == m0, idx, 128), axis=1, keepdims=True)
    l2 = jnp.where(idx == e0, jnp.finfo(jnp.float32).min, logits)
    m1 = jnp.max(l2, axis=1, keepdims=True)
    e1 = jnp.min(jnp.where(l2 == m1, idx, 128), axis=1, keepdims=True)
    d = jnp.exp(m1 - m0)
    s = 1.0 + d
    e_ref[...] = jnp.concatenate([e0, e1], axis=1)
    w_ref[...] = jnp.concatenate([1.0 / s, d / s], axis=1)


def _gate(x, wg_pad, bg_pad):
    return pl.pallas_call(
        _gate_body,
        grid=(T // GT,),
        in_specs=[
            pl.BlockSpec((GT, D), lambda i: (i, 0)),
            pl.BlockSpec((D, 128), lambda i: (0, 0)),
            pl.BlockSpec((1, 128), lambda i: (0, 0)),
        ],
        out_specs=[
            pl.BlockSpec((GT, 2), lambda i: (i, 0)),
            pl.BlockSpec((GT, 2), lambda i: (i, 0)),
            pl.BlockSpec((GT, D // 2), lambda i: (i, 0)),
        ],
        out_shape=[
            jax.ShapeDtypeStruct((T, 2), jnp.int32),
            jax.ShapeDtypeStruct((T, 2), jnp.float32),
            jax.ShapeDtypeStruct((T, D // 2), jnp.int32),
        ],
    )(x, wg_pad, bg_pad)


def _cumsum0(x):
    n = x.shape[0]
    d = 1
    while d < n:
        pad = jnp.zeros((d,) + x.shape[1:], x.dtype)
        x = x + jnp.concatenate([pad, x[:-d]], axis=0)
        d *= 2
    return x


def _cumsum1(x):
    n = x.shape[1]
    d = 1
    while d < n:
        pad = jnp.zeros(x.shape[:1] + (d,), x.dtype)
        x = x + jnp.concatenate([pad, x[:, :-d]], axis=1)
        d *= 2
    return x


def _meta_body(e_ref, w_ref, pos_ref, wb_ref, itile_ref, ie_ref,
               offs_ref, tot_ref):
    lane8 = lax.broadcasted_iota(jnp.int32, (T, E), 1)
    e0 = e_ref[:, 0:1]
    e1 = e_ref[:, 1:2]
    oh_a = (e0 == lane8).astype(jnp.int32)
    oh_b = (e1 == lane8).astype(jnp.int32)
    c_a = _cumsum0(oh_a)
    c_b = _cumsum0(oh_b)
    rank_a = jnp.sum(oh_a * (c_a + c_b - oh_b), axis=1, keepdims=True) - 1
    rank_b = jnp.sum(oh_b * (c_a + c_b), axis=1, keepdims=True) - 1
    counts = c_a[T - 1:T, :] + c_b[T - 1:T, :]
    offs_ex = _cumsum1(counts) - counts
    off_a = jnp.sum(oh_a * offs_ex, axis=1, keepdims=True)
    off_b = jnp.sum(oh_b * offs_ex, axis=1, keepdims=True)
    pos_ref[...] = jnp.concatenate([off_a + rank_a, off_b + rank_b], axis=1)
    w = w_ref[...]
    wb_ref[...] = jnp.concatenate(
        [jnp.broadcast_to(w[:, 0:1], (T, 16)),
         jnp.broadcast_to(w[:, 1:2], (T, 16))], axis=1)

    ends = offs_ex + counts
    start_tile = offs_ex // BT
    last_tile = jnp.maximum(ends - 1, 0) // BT
    n_items = jnp.where(counts > 0, last_tile - start_tile + 1, 0)
    cum_it = _cumsum1(n_items)
    cum_ex = cum_it - n_items
    total = cum_it[0:1, E - 1:E]
    ii = lax.broadcasted_iota(jnp.int32, (1, 128), 1)
    ge = (lax.broadcasted_iota(jnp.int32, (E, 128), 1) >=
          jnp.broadcast_to(cum_it.reshape(E, 1), (E, 128))).astype(jnp.int32)
    eid = jnp.sum(ge, axis=0, keepdims=True)
    sel = (lax.broadcasted_iota(jnp.int32, (E, 128), 0) ==
           jnp.broadcast_to(eid, (E, 128))).astype(jnp.int32)
    st = jnp.sum(sel * start_tile.reshape(E, 1), axis=0, keepdims=True)
    cx = jnp.sum(sel * cum_ex.reshape(E, 1), axis=0, keepdims=True)
    tile_i = st + (ii - cx)
    real = ii < total
    lane8r = lax.broadcasted_iota(jnp.int32, (1, E), 1)
    e_last = jnp.max(jnp.where(counts > 0, lane8r, 0))
    itile_ref[...] = jnp.where(real, tile_i, NT - 1)
    ie_ref[...] = jnp.where(real, jnp.minimum(eid, E - 1), e_last)
    offs_ref[...] = jnp.concatenate(
        [offs_ex, jnp.full((1, 16 - E), P, jnp.int32)], axis=1)
    tot_ref[...] = jnp.broadcast_to(total, (1, 8))


def _meta(e01, w01):
    return pl.pallas_call(
        _meta_body,
        grid=(1,),
        in_specs=[
            pl.BlockSpec((T, 2), lambda i: (0, 0)),
            pl.BlockSpec((T, 2), lambda i: (0, 0)),
        ],
        out_specs=[
            pl.BlockSpec((T, 2), lambda i: (0, 0)),
            pl.BlockSpec((T, 32), lambda i: (0, 0)),
            pl.BlockSpec((1, 128), lambda i: (0, 0)),
            pl.BlockSpec((1, 128), lambda i: (0, 0)),
            pl.BlockSpec((1, 16), lambda i: (0, 0)),
            pl.BlockSpec((1, 8), lambda i: (0, 0)),
        ],
        out_shape=[
            jax.ShapeDtypeStruct((T, 2), jnp.int32),
            jax.ShapeDtypeStruct((T, 32), jnp.float32),
            jax.ShapeDtypeStruct((1, 128), jnp.int32),
            jax.ShapeDtypeStruct((1, 128), jnp.int32),
            jax.ShapeDtypeStruct((1, 16), jnp.int32),
            jax.ShapeDtypeStruct((1, 8), jnp.int32),
        ],
    )(e01, w01)


_CHUNK = 32
_SC_MESH = dict(core_axis_name="c", subcore_axis_name="s")


def _permute_body(x_hbm, pos_hbm, xs_hbm, tokv, posv, buf, sem, sem2):
    wid = lax.axis_index("s") * 2 + lax.axis_index("c")
    base = wid * (P // NW)
    iot = lax.iota(jnp.int32, 16)

    def body(b, carry):
        j0 = base + b * _CHUNK
        pltpu.sync_copy(pos_hbm.at[pl.ds(j0, _CHUNK)], posv)
        tokv[pl.ds(0, 16)] = lax.shift_right_logical(j0 + iot, 1)
        tokv[pl.ds(16, 16)] = lax.shift_right_logical(j0 + 16 + iot, 1)
        pltpu.async_copy(x_hbm.at[tokv], buf, sem).wait()
        pltpu.async_copy(buf, xs_hbm.at[posv], sem2).wait()
        return carry

    lax.fori_loop(0, (P // NW) // _CHUNK, body, 0)


def _permute(x, pos_flat):
    k = functools.partial(
        pl.kernel,
        mesh=plsc.VectorSubcoreMesh(**_SC_MESH),
        out_type=jax.ShapeDtypeStruct((P, D // 2), jnp.int32),
        scratch_types=[
            pltpu.VMEM((_CHUNK,), jnp.int32),
            pltpu.VMEM((_CHUNK,), jnp.int32),
            pltpu.VMEM((_CHUNK, D // 2), jnp.int32),
            pltpu.SemaphoreType.DMA,
            pltpu.SemaphoreType.DMA,
        ],
    )(_permute_body)
    return k(x, pos_flat)


def _row_mask(itile_s, ie_s, offs_s, i):
    tile = itile_s[i]
    e = ie_s[i]
    lo = offs_s[e]
    hi = offs_s[e + 1]
    r = tile * BT + lax.broadcasted_iota(jnp.int32, (BT, 1), 0)
    return jnp.logical_and(r >= lo, r < hi)


def _init_cond(itile_s, i):
    prev = itile_s[jnp.maximum(i - 1, 0)]
    return jnp.logical_or(i == 0, itile_s[i] != prev)


def _mlp_a_body(itile_s, ie_s, offs_s, tot_s, xs_ref, w1_ref, b1_ref, h_ref):
    i = pl.program_id(0)

    @pl.when(_init_cond(itile_s, i))
    def _():
        h_ref[...] = jnp.zeros_like(h_ref)

    @pl.when(i < tot_s[0])
    def _():
        xi = xs_ref[...]
        lo = lax.bitcast_convert_type(
            lax.shift_left(xi, 16), jnp.float32).astype(jnp.bfloat16)
        hi = lax.bitcast_convert_type(
            jnp.bitwise_and(xi, jnp.int32(-65536)),
            jnp.float32).astype(jnp.bfloat16)
        x = jnp.concatenate([lo, hi], axis=1)
        h = lax.dot_general(x, w1_ref[0], (((1,), (0,)), ((), ())),
                            preferred_element_type=jnp.float32)
        h = h + b1_ref[0]
        h = h / (1.0 + jnp.exp(-h))
        mask = _row_mask(itile_s, ie_s, offs_s, i)
        h_ref[...] += jnp.where(mask, h, 0.0).astype(jnp.bfloat16)


def _mlp_b_body(itile_s, ie_s, offs_s, tot_s, h_ref, w2_ref, b2_ref, out_ref):
    i = pl.program_id(0)

    @pl.when(_init_cond(itile_s, i))
    def _():
        out_ref[...] = jnp.zeros_like(out_ref)

    @pl.when(i < tot_s[0])
    def _():
        y = lax.dot_general(h_ref[...], w2_ref[0], (((1,), (0,)), ((), ())),
                            preferred_element_type=jnp.float32)
        y = y + b2_ref[0]
        bits = lax.bitcast_convert_type(
            y.astype(jnp.bfloat16).astype(jnp.float32), jnp.int32)
        lo = lax.shift_right_logical(bits[:, :D // 2], 16)
        hi = jnp.bitwise_and(bits[:, D // 2:], jnp.int32(-65536))
        packed = jnp.bitwise_or(lo, hi)
        mask = _row_mask(itile_s, ie_s, offs_s, i)
        out_ref[...] += jnp.where(mask, packed, 0)


def _mlp(itile, ie, offs16, tot, xs, w1bf, b1, w2bf, b2):
    def spec(i_map_shapes):
        return pltpu.PrefetchScalarGridSpec(
            num_scalar_prefetch=4,
            grid=(NI,),
            in_specs=[
                pl.BlockSpec(i_map_shapes[0],
                             lambda i, it, ie_, of, tt: (it[i], 0)),
                pl.BlockSpec(i_map_shapes[1],
                             lambda i, it, ie_, of, tt: (ie_[i], 0, 0)),
                pl.BlockSpec(i_map_shapes[2],
                             lambda i, it, ie_, of, tt: (ie_[i], 0, 0)),
            ],
            out_specs=pl.BlockSpec(i_map_shapes[3],
                                   lambda i, it, ie_, of, tt: (it[i], 0)),
        )

    h = pl.pallas_call(
        _mlp_a_body,
        grid_spec=spec([(BT, D // 2), (1, D, F), (1, 1, F), (BT, F)]),
        out_shape=jax.ShapeDtypeStruct((P, F), jnp.bfloat16),
        compiler_params=pltpu.CompilerParams(
            vmem_limit_bytes=60 * 1024 * 1024),
    )(itile, ie, offs16, tot, xs, w1bf, b1.reshape(E, 1, F))
    return pl.pallas_call(
        _mlp_b_body,
        grid_spec=spec([(BT, F), (1, F, D), (1, 1, D), (BT, D // 2)]),
        out_shape=jax.ShapeDtypeStruct((P, D // 2), jnp.int32),
        compiler_params=pltpu.CompilerParams(
            vmem_limit_bytes=60 * 1024 * 1024),
    )(itile, ie, offs16, tot, h, w2bf, b2.reshape(E, 1, D))


def _gatherj_body(ys_hbm, pos_hbm, yg_hbm, posv, buf, sem):
    wid = lax.axis_index("s") * 2 + lax.axis_index("c")
    base = wid * (P // NW)

    def body(b, carry):
        j0 = base + b * _CHUNK
        pltpu.sync_copy(pos_hbm.at[pl.ds(j0, _CHUNK)], posv)
        pltpu.async_copy(ys_hbm.at[posv], buf, sem).wait()
        pltpu.sync_copy(buf, yg_hbm.at[pl.ds(j0, _CHUNK)])
        return carry

    lax.fori_loop(0, (P // NW) // _CHUNK, body, 0)


def _gatherj(ys_packed, pos_flat):
    k = functools.partial(
        pl.kernel,
        mesh=plsc.VectorSubcoreMesh(**_SC_MESH),
        out_type=jax.ShapeDtypeStruct((P, D // 2), jnp.int32),
        scratch_types=[
            pltpu.VMEM((_CHUNK,), jnp.int32),
            pltpu.VMEM((_CHUNK, D // 2), jnp.int32),
            pltpu.SemaphoreType.DMA,
        ],
    )(_gatherj_body)
    return k(ys_packed, pos_flat)


BTC = 512


def _combine_body(yg_ref, wb_ref, out_ref):
    yi = yg_ref[...]
    lo = lax.bitcast_convert_type(lax.shift_left(yi, 16), jnp.float32)
    hi = lax.bitcast_convert_type(
        jnp.bitwise_and(yi, jnp.int32(-65536)), jnp.float32)
    y = jnp.concatenate([lo, hi], axis=1)
    s = y * wb_ref[:, 0:1]
    s3 = s.reshape(BTC, 2, D)
    out_ref[...] = s3[:, 0, :] + s3[:, 1, :]


def _combine(yg, wb):
    return pl.pallas_call(
        _combine_body,
        grid=(T // BTC,),
        in_specs=[
            pl.BlockSpec((2 * BTC, D // 2), lambda i: (i, 0)),
            pl.BlockSpec((2 * BTC, 16), lambda i: (i, 0)),
        ],
        out_specs=pl.BlockSpec((BTC, D), lambda i: (i, 0)),
        out_shape=jax.ShapeDtypeStruct((T, D), jnp.float32),
    )(yg, wb)


def kernel(inputs, Wg, bg, W1, b1, W2, b2):
    x = inputs
    wg_pad = jnp.zeros((D, 128), jnp.float32).at[:, :E].set(Wg)
    bg_pad = jnp.full((1, 128), -1e30, jnp.float32).at[0, :E].set(bg)
    e01, w01, xi = _gate(x, wg_pad, bg_pad)
    pos2, wb32, itile, ie, offs16, tot = _meta(e01, w01)
    pos_flat = pos2.reshape(P)
    wb = wb32.reshape(P, 16)
    xs = _permute(xi, pos_flat)
    ys = _mlp(itile.reshape(128), ie.reshape(128), offs16.reshape(16),
              tot.reshape(8), xs, W1.astype(jnp.bfloat16), b1,
              W2.astype(jnp.bfloat16), b2)
    yg = _gatherj(ys, pos_flat)
    return _combine(yg, wb)


# speedup vs baseline: 1.0157x; 1.0157x over previous
import functools

import jax
import jax.numpy as jnp
from jax import lax
from jax.experimental import pallas as pl
from jax.experimental.pallas import tpu as pltpu
from jax.experimental.pallas import tpu_sc as plsc

E = 8
D = 2048
F = 4096
T = 8192
P = 2 * T
BT = 256
NT = P // BT
NI = NT + E - 1
NW = 32
GT = 1024


def _gate_body(x_ref, wg_ref, bg_ref, e_ref, w_ref, xi_ref):
    x = x_ref[...].astype(jnp.bfloat16)
    wg = wg_ref[...].astype(jnp.bfloat16)
    logits = lax.dot_general(x, wg, (((1,), (0,)), ((), ())),
                             preferred_element_type=jnp.float32)
    logits = logits + bg_ref[...]
    bits = lax.bitcast_convert_type(x.astype(jnp.float32), jnp.int32)
    lo = lax.shift_right_logical(bits[:, :D // 2], 16)
    hi = jnp.bitwise_and(bits[:, D // 2:], jnp.int32(-65536))
    xi_ref[...] = jnp.bitwise_or(lo, hi)
    idx = lax.broadcasted_iota(jnp.int32, logits.shape, 1)
    m0 = jnp.max(logits, axis=1, keepdims=True)
    e0 = jnp.min(jnp.where(logits == m0, idx, 128), axis=1, keepdims=True)
    l2 = jnp.where(idx == e0, jnp.finfo(jnp.float32).min, logits)
    m1 = jnp.max(l2, axis=1, keepdims=True)
    e1 = jnp.min(jnp.where(l2 == m1, idx, 128), axis=1, keepdims=True)
    d = jnp.exp(m1 - m0)
    s = 1.0 + d
    e_ref[...] = jnp.concatenate([e0, e1], axis=1)
    w_ref[...] = jnp.concatenate([1.0 / s, d / s], axis=1)


def _gate(x, wg_pad, bg_pad):
    return pl.pallas_call(
        _gate_body,
        grid=(T // GT,),
        in_specs=[
            pl.BlockSpec((GT, D), lambda i: (i, 0)),
            pl.BlockSpec((D, 128), lambda i: (0, 0)),
            pl.BlockSpec((1, 128), lambda i: (0, 0)),
        ],
        out_specs=[
            pl.BlockSpec((GT, 2), lambda i: (i, 0)),
            pl.BlockSpec((GT, 2), lambda i: (i, 0)),
            pl.BlockSpec((GT, D // 2), lambda i: (i, 0)),
        ],
        out_shape=[
            jax.ShapeDtypeStruct((T, 2), jnp.int32),
            jax.ShapeDtypeStruct((T, 2), jnp.float32),
            jax.ShapeDtypeStruct((T, D // 2), jnp.int32),
        ],
    )(x, wg_pad, bg_pad)


def _cumsum0(x):
    n = x.shape[0]
    d = 1
    while d < n:
        pad = jnp.zeros((d,) + x.shape[1:], x.dtype)
        x = x + jnp.concatenate([pad, x[:-d]], axis=0)
        d *= 2
    return x


def _cumsum1(x):
    n = x.shape[1]
    d = 1
    while d < n:
        pad = jnp.zeros(x.shape[:1] + (d,), x.dtype)
        x = x + jnp.concatenate([pad, x[:, :-d]], axis=1)
        d *= 2
    return x


def _meta_body(e_ref, w_ref, pos_ref, wb_ref, itile_ref, ie_ref,
               offs_ref, tot_ref):
    lane8 = lax.broadcasted_iota(jnp.int32, (T, E), 1)
    e0 = e_ref[:, 0:1]
    e1 = e_ref[:, 1:2]
    oh_a = (e0 == lane8).astype(jnp.int32)
    oh_b = (e1 == lane8).astype(jnp.int32)
    c_a = _cumsum0(oh_a)
    c_b = _cumsum0(oh_b)
    rank_a = jnp.sum(oh_a * (c_a + c_b - oh_b), axis=1, keepdims=True) - 1
    rank_b = jnp.sum(oh_b * (c_a + c_b), axis=1, keepdims=True) - 1
    counts = c_a[T - 1:T, :] + c_b[T - 1:T, :]
    offs_ex = _cumsum1(counts) - counts
    off_a = jnp.sum(oh_a * offs_ex, axis=1, keepdims=True)
    off_b = jnp.sum(oh_b * offs_ex, axis=1, keepdims=True)
    pos_ref[...] = jnp.concatenate([off_a + rank_a, off_b + rank_b], axis=1)
    w = w_ref[...]
    wb_ref[...] = jnp.concatenate(
        [jnp.broadcast_to(w[:, 0:1], (T, 16)),
         jnp.broadcast_to(w[:, 1:2], (T, 16))], axis=1)

    ends = offs_ex + counts
    start_tile = offs_ex // BT
    last_tile = jnp.maximum(ends - 1, 0) // BT
    n_items = jnp.where(counts > 0, last_tile - start_tile + 1, 0)
    cum_it = _cumsum1(n_items)
    cum_ex = cum_it - n_items
    total = cum_it[0:1, E - 1:E]
    ii = lax.broadcasted_iota(jnp.int32, (1, 128), 1)
    ge = (lax.broadcasted_iota(jnp.int32, (E, 128), 1) >=
          jnp.broadcast_to(cum_it.reshape(E, 1), (E, 128))).astype(jnp.int32)
    eid = jnp.sum(ge, axis=0, keepdims=True)
    sel = (lax.broadcasted_iota(jnp.int32, (E, 128), 0) ==
           jnp.broadcast_to(eid, (E, 128))).astype(jnp.int32)
    st = jnp.sum(sel * start_tile.reshape(E, 1), axis=0, keepdims=True)
    cx = jnp.sum(sel * cum_ex.reshape(E, 1), axis=0, keepdims=True)
    tile_i = st + (ii - cx)
    real = ii < total
    lane8r = lax.broadcasted_iota(jnp.int32, (1, E), 1)
    e_last = jnp.max(jnp.where(counts > 0, lane8r, 0))
    itile_ref[...] = jnp.where(real, tile_i, NT - 1)
    ie_ref[...] = jnp.where(real, jnp.minimum(eid, E - 1), e_last)
    offs_ref[...] = jnp.concatenate(
        [offs_ex, jnp.full((1, 16 - E), P, jnp.int32)], axis=1)
    tot_ref[...] = jnp.broadcast_to(total, (1, 8))


def _meta(e01, w01):
    return pl.pallas_call(
        _meta_body,
        grid=(1,),
        in_specs=[
            pl.BlockSpec((T, 2), lambda i: (0, 0)),
            pl.BlockSpec((T, 2), lambda i: (0, 0)),
        ],
        out_specs=[
            pl.BlockSpec((T, 2), lambda i: (0, 0)),
            pl.BlockSpec((T, 32), lambda i: (0, 0)),
            pl.BlockSpec((1, 128), lambda i: (0, 0)),
            pl.BlockSpec((1, 128), lambda i: (0, 0)),
            pl.BlockSpec((1, 16), lambda i: (0, 0)),
            pl.BlockSpec((1, 8), lambda i: (0, 0)),
        ],
        out_shape=[
            jax.ShapeDtypeStruct((T, 2), jnp.int32),
            jax.ShapeDtypeStruct((T, 32), jnp.float32),
            jax.ShapeDtypeStruct((1, 128), jnp.int32),
            jax.ShapeDtypeStruct((1, 128), jnp.int32),
            jax.ShapeDtypeStruct((1, 16), jnp.int32),
            jax.ShapeDtypeStruct((1, 8), jnp.int32),
        ],
    )(e01, w01)


_CHUNK = 32
_SC_MESH = dict(core_axis_name="c", subcore_axis_name="s")


_NB = (P // NW) // _CHUNK


def _permute_body(x_hbm, pos3_hbm, xs_hbm, tokv0, tokv1, posv2,
                  buf0, buf1, sg0, sg1, ss0, ss1):
    wid = lax.axis_index("s") * 2 + lax.axis_index("c")
    base = wid * (P // NW)
    iot = lax.iota(jnp.int32, 16)
    pltpu.sync_copy(pos3_hbm.at[wid], posv2)
    tokv = (tokv0, tokv1)
    buf = (buf0, buf1)
    sg = (sg0, sg1)
    ss = (ss0, ss1)

    def issue_gather(b):
        j0 = base + b * _CHUNK
        tv = tokv[b % 2]
        tv[pl.ds(0, 16)] = lax.shift_right_logical(j0 + iot, 1)
        tv[pl.ds(16, 16)] = lax.shift_right_logical(j0 + 16 + iot, 1)
        return pltpu.async_copy(x_hbm.at[tv], buf[b % 2], sg[b % 2])

    g = issue_gather(0)
    scat = [None, None]
    for b in range(_NB):
        if b + 1 < _NB:
            if scat[(b + 1) % 2] is not None:
                scat[(b + 1) % 2].wait()
            g_next = issue_gather(b + 1)
        g.wait()
        scat[b % 2] = pltpu.async_copy(
            buf[b % 2], xs_hbm.at[posv2.at[b]], ss[b % 2])
        if b + 1 < _NB:
            g = g_next
    scat[(_NB - 1) % 2].wait()
    scat[_NB % 2].wait()


def _permute(x, pos3):
    k = functools.partial(
        pl.kernel,
        mesh=plsc.VectorSubcoreMesh(**_SC_MESH),
        out_type=jax.ShapeDtypeStruct((P, D // 2), jnp.int32),
        scratch_types=[
            pltpu.VMEM((_CHUNK,), jnp.int32),
            pltpu.VMEM((_CHUNK,), jnp.int32),
            pltpu.VMEM((_NB, _CHUNK), jnp.int32),
            pltpu.VMEM((_CHUNK, D // 2), jnp.int32),
            pltpu.VMEM((_CHUNK, D // 2), jnp.int32),
            pltpu.SemaphoreType.DMA,
            pltpu.SemaphoreType.DMA,
            pltpu.SemaphoreType.DMA,
            pltpu.SemaphoreType.DMA,
        ],
    )(_permute_body)
    return k(x, pos3)


def _row_mask(itile_s, ie_s, offs_s, i):
    tile = itile_s[i]
    e = ie_s[i]
    lo = offs_s[e]
    hi = offs_s[e + 1]
    r = tile * BT + lax.broadcasted_iota(jnp.int32, (BT, 1), 0)
    return jnp.logical_and(r >= lo, r < hi)


def _init_cond(itile_s, i):
    prev = itile_s[jnp.maximum(i - 1, 0)]
    return jnp.logical_or(i == 0, itile_s[i] != prev)


def _mlp_a_body(itile_s, ie_s, offs_s, tot_s, xs_ref, w1_ref, b1_ref, h_ref):
    i = pl.program_id(0)

    @pl.when(_init_cond(itile_s, i))
    def _():
        h_ref[...] = jnp.zeros_like(h_ref)

    @pl.when(i < tot_s[0])
    def _():
        xi = xs_ref[...]
        lo = lax.bitcast_convert_type(
            lax.shift_left(xi, 16), jnp.float32).astype(jnp.bfloat16)
        hi = lax.bitcast_convert_type(
            jnp.bitwise_and(xi, jnp.int32(-65536)),
            jnp.float32).astype(jnp.bfloat16)
        x = jnp.concatenate([lo, hi], axis=1)
        h = lax.dot_general(x, w1_ref[0], (((1,), (0,)), ((), ())),
                            preferred_element_type=jnp.float32)
        h = h + b1_ref[0]
        h = h / (1.0 + jnp.exp(-h))
        mask = _row_mask(itile_s, ie_s, offs_s, i)
        h_ref[...] += jnp.where(mask, h, 0.0).astype(jnp.bfloat16)


def _mlp_b_body(itile_s, ie_s, offs_s, tot_s, h_ref, w2_ref, b2_ref, out_ref):
    i = pl.program_id(0)

    @pl.when(_init_cond(itile_s, i))
    def _():
        out_ref[...] = jnp.zeros_like(out_ref)

    @pl.when(i < tot_s[0])
    def _():
        y = lax.dot_general(h_ref[...], w2_ref[0], (((1,), (0,)), ((), ())),
                            preferred_element_type=jnp.float32)
        y = y + b2_ref[0]
        bits = lax.bitcast_convert_type(
            y.astype(jnp.bfloat16).astype(jnp.float32), jnp.int32)
        lo = lax.shift_right_logical(bits[:, :D // 2], 16)
        hi = jnp.bitwise_and(bits[:, D // 2:], jnp.int32(-65536))
        packed = jnp.bitwise_or(lo, hi)
        mask = _row_mask(itile_s, ie_s, offs_s, i)
        out_ref[...] += jnp.where(mask, packed, 0)


def _mlp(itile, ie, offs16, tot, xs, w1bf, b1, w2bf, b2):
    def spec(i_map_shapes):
        return pltpu.PrefetchScalarGridSpec(
            num_scalar_prefetch=4,
            grid=(NI,),
            in_specs=[
                pl.BlockSpec(i_map_shapes[0],
                             lambda i, it, ie_, of, tt: (it[i], 0)),
                pl.BlockSpec(i_map_shapes[1],
                             lambda i, it, ie_, of, tt: (ie_[i], 0, 0)),
                pl.BlockSpec(i_map_shapes[2],
                             lambda i, it, ie_, of, tt: (ie_[i], 0, 0)),
            ],
            out_specs=pl.BlockSpec(i_map_shapes[3],
                                   lambda i, it, ie_, of, tt: (it[i], 0)),
        )

    h = pl.pallas_call(
        _mlp_a_body,
        grid_spec=spec([(BT, D // 2), (1, D, F), (1, 1, F), (BT, F)]),
        out_shape=jax.ShapeDtypeStruct((P, F), jnp.bfloat16),
        compiler_params=pltpu.CompilerParams(
            vmem_limit_bytes=60 * 1024 * 1024),
    )(itile, ie, offs16, tot, xs, w1bf, b1.reshape(E, 1, F))
    return pl.pallas_call(
        _mlp_b_body,
        grid_spec=spec([(BT, F), (1, F, D), (1, 1, D), (BT, D // 2)]),
        out_shape=jax.ShapeDtypeStruct((P, D // 2), jnp.int32),
        compiler_params=pltpu.CompilerParams(
            vmem_limit_bytes=60 * 1024 * 1024),
    )(itile, ie, offs16, tot, h, w2bf, b2.reshape(E, 1, D))


def _gatherj_body(ys_hbm, pos3_hbm, yg_hbm, posv2,
                  buf0, buf1, sg0, sg1, ss0, ss1):
    wid = lax.axis_index("s") * 2 + lax.axis_index("c")
    base = wid * (P // NW)
    pltpu.sync_copy(pos3_hbm.at[wid], posv2)
    buf = (buf0, buf1)
    sg = (sg0, sg1)
    ss = (ss0, ss1)

    def issue_gather(b):
        return pltpu.async_copy(
            ys_hbm.at[posv2.at[b]], buf[b % 2], sg[b % 2])

    g = issue_gather(0)
    st = [None, None]
    for b in range(_NB):
        if b + 1 < _NB:
            if st[(b + 1) % 2] is not None:
                st[(b + 1) % 2].wait()
            g_next = issue_gather(b + 1)
        g.wait()
        st[b % 2] = pltpu.async_copy(
            buf[b % 2], yg_hbm.at[pl.ds(base + b * _CHUNK, _CHUNK)],
            ss[b % 2])
        if b + 1 < _NB:
            g = g_next
    st[(_NB - 1) % 2].wait()
    st[_NB % 2].wait()


def _gatherj(ys_packed, pos3):
    k = functools.partial(
        pl.kernel,
        mesh=plsc.VectorSubcoreMesh(**_SC_MESH),
        out_type=jax.ShapeDtypeStruct((P, D // 2), jnp.int32),
        scratch_types=[
            pltpu.VMEM((_NB, _CHUNK), jnp.int32),
            pltpu.VMEM((_CHUNK, D // 2), jnp.int32),
            pltpu.VMEM((_CHUNK, D // 2), jnp.int32),
            pltpu.SemaphoreType.DMA,
            pltpu.SemaphoreType.DMA,
            pltpu.SemaphoreType.DMA,
            pltpu.SemaphoreType.DMA,
        ],
    )(_gatherj_body)
    return k(ys_packed, pos3)


BTC = 512


def _combine_body(yg_ref, wb_ref, out_ref):
    yi = yg_ref[...]
    lo = lax.bitcast_convert_type(lax.shift_left(yi, 16), jnp.float32)
    hi = lax.bitcast_convert_type(
        jnp.bitwise_and(yi, jnp.int32(-65536)), jnp.float32)
    y = jnp.concatenate([lo, hi], axis=1)
    s = y * wb_ref[:, 0:1]
    s3 = s.reshape(BTC, 2, D)
    out_ref[...] = s3[:, 0, :] + s3[:, 1, :]


def _combine(yg, wb):
    return pl.pallas_call(
        _combine_body,
        grid=(T // BTC,),
        in_specs=[
            pl.BlockSpec((2 * BTC, D // 2), lambda i: (i, 0)),
            pl.BlockSpec((2 * BTC, 16), lambda i: (i, 0)),
        ],
        out_specs=pl.BlockSpec((BTC, D), lambda i: (i, 0)),
        out_shape=jax.ShapeDtypeStruct((T, D), jnp.float32),
    )(yg, wb)


def kernel(inputs, Wg, bg, W1, b1, W2, b2):
    x = inputs
    wg_pad = jnp.zeros((D, 128), jnp.float32).at[:, :E].set(Wg)
    bg_pad = jnp.full((1, 128), -1e30, jnp.float32).at[0, :E].set(bg)
    e01, w01, xi = _gate(x, wg_pad, bg_pad)
    pos2, wb32, itile, ie, offs16, tot = _meta(e01, w01)
    pos3 = pos2.reshape(NW, _NB, _CHUNK)
    wb = wb32.reshape(P, 16)
    xs = _permute(xi, pos3)
    ys = _mlp(itile.reshape(128), ie.reshape(128), offs16.reshape(16),
              tot.reshape(8), xs, W1.astype(jnp.bfloat16), b1,
              W2.astype(jnp.bfloat16), b2)
    yg = _gatherj(ys, pos3)
    return _combine(yg, wb)
